# Initial kernel scaffold; baseline (speedup 1.0000x reference)
import functools

import jax
import jax.numpy as jnp
from jax import lax
from jax.experimental import pallas as pl
from jax.experimental.pallas import tpu as pltpu
from jax.experimental.pallas import tpu_sc as plsc

_K = 8192
_D = 32
_N = 8192
_R = 256
_G = _N // _R
_BETA = 0.25

_NC, _NS = 2, 16
_NW = _NC * _NS
_RW = _N // _NW
_IC = 128
_NCHUNK = _RW // _IC


def _tc_body(x_ref, a_ref, b_ref, e_ref, idx_ref, acc_ref):
    x = x_ref[...]
    e = e_ref[...]
    m = lax.dot_general(x, e, (((1,), (1,)), ((), ())),
                        preferred_element_type=jnp.float32)
    dist = (a_ref[...] + b_ref[...]) - 2.0 * m
    idx = jnp.argmin(dist, axis=1).astype(jnp.int32)
    dmin = jnp.min(dist, axis=1)
    idx_ref[...] = idx.reshape(1, 1, _R)

    @pl.when(pl.program_id(0) == 0)
    def _():
        acc_ref[0, 0] = jnp.float32(0.0)

    acc_ref[0, 0] += jnp.sum(dmin)


_tc_call = pl.pallas_call(
    _tc_body,
    grid=(_G,),
    in_specs=[
        pl.BlockSpec((_R, _D), lambda i: (i, 0)),
        pl.BlockSpec((_R, 1), lambda i: (i, 0)),
        pl.BlockSpec((1, _K), lambda i: (0, 0)),
        pl.BlockSpec((_K, _D), lambda i: (0, 0)),
    ],
    out_specs=[
        pl.BlockSpec((1, 1, _R), lambda i: (i, 0, 0)),
        pl.BlockSpec((1, 1), lambda i: (0, 0)),
    ],
    out_shape=[
        jax.ShapeDtypeStruct((_G, 1, _R), jnp.int32),
        jax.ShapeDtypeStruct((1, 1), jnp.float32),
    ],
)


_sc_mesh = plsc.VectorSubcoreMesh(core_axis_name="c", subcore_axis_name="s")


@functools.partial(
    pl.kernel,
    out_type=jax.ShapeDtypeStruct((_N, _D), jnp.float32),
    mesh=_sc_mesh,
    scratch_types=[
        pltpu.VMEM((_NCHUNK, _IC), jnp.int32),
        pltpu.VMEM((_RW, _D), jnp.float32),
        pltpu.VMEM((_RW, _D), jnp.float32),
        pltpu.SemaphoreType.DMA,
    ],
)
def _sc_gather(e_hbm, idx_hbm, x_hbm, out_hbm, idx_v, q_v, x_v, sem):
    wid = lax.axis_index("s") * _NC + lax.axis_index("c")
    base = wid * _RW
    pltpu.sync_copy(idx_hbm.at[pl.ds(wid * _NCHUNK, _NCHUNK), :], idx_v)
    pltpu.sync_copy(x_hbm.at[pl.ds(base, _RW), :], x_v)
    copies = []
    for c in range(_NCHUNK):
        copies.append(pltpu.async_copy(
            e_hbm.at[idx_v.at[c]], q_v.at[pl.ds(c * _IC, _IC), :], sem))
    for cp in copies:
        cp.wait()

    def body(i, _):
        xa = x_v[i, pl.ds(0, 16)]
        qa = q_v[i, pl.ds(0, 16)]
        q_v[i, pl.ds(0, 16)] = xa + (qa - xa)
        xb = x_v[i, pl.ds(16, 16)]
        qb = q_v[i, pl.ds(16, 16)]
        q_v[i, pl.ds(16, 16)] = xb + (qb - xb)
        return 0

    lax.fori_loop(0, _RW, body, 0)
    pltpu.sync_copy(q_v, out_hbm.at[pl.ds(base, _RW), :])


def kernel(inputs, embedding_weight):
    x = jnp.transpose(inputs, (0, 2, 3, 1)).reshape(_N, _D)
    a = jnp.sum(x ** 2, axis=1, keepdims=True)
    b = jnp.sum(embedding_weight ** 2, axis=1).reshape(1, _K)
    idx3, acc = _tc_call(x, a, b, embedding_weight)
    idx = idx3.reshape(_NW * _NCHUNK, _IC)
    out_flat = _sc_gather(embedding_weight, idx, x)
    loss = (1.0 + _BETA) * acc[0, 0] / jnp.float32(_N * _D)
    out = out_flat.reshape(8, 32, 32, _D).transpose(0, 3, 1, 2)
    return (loss, out)


# speedup vs baseline: 3.1934x; 3.1934x over previous
import functools

import jax
import jax.numpy as jnp
from jax import lax
from jax.experimental import pallas as pl
from jax.experimental.pallas import tpu as pltpu
from jax.experimental.pallas import tpu_sc as plsc

_K = 8192
_D = 32
_N = 8192
_R = 256
_G = _N // _R
_BETA = 0.25

_NC, _NS = 2, 16
_NW = _NC * _NS
_RW = _N // _NW
_IC = 128
_NCHUNK = _RW // _IC


def _tc_body(x_ref, a_ref, b_ref, e_ref, idx_ref, acc_ref):
    x = x_ref[...]
    e = e_ref[...]
    m = lax.dot_general(x, e, (((1,), (1,)), ((), ())),
                        preferred_element_type=jnp.float32)
    dist = (a_ref[...] + b_ref[...]) - 2.0 * m
    dmin2 = jnp.min(dist, axis=1, keepdims=True)
    ii = lax.broadcasted_iota(jnp.int32, (_R, _K), 1)
    idx = jnp.min(jnp.where(dist == dmin2, ii, _K), axis=1).astype(jnp.int32)
    dmin = dmin2.reshape(_R)
    idx_ref[...] = idx.reshape(1, 1, _R)

    @pl.when(pl.program_id(0) == 0)
    def _():
        acc_ref[...] = jnp.zeros((1, 1), jnp.float32)

    acc_ref[...] += jnp.sum(dmin).reshape(1, 1)


_tc_call = pl.pallas_call(
    _tc_body,
    grid=(_G,),
    in_specs=[
        pl.BlockSpec((_R, _D), lambda i: (i, 0)),
        pl.BlockSpec((_R, 1), lambda i: (i, 0)),
        pl.BlockSpec((1, _K), lambda i: (0, 0)),
        pl.BlockSpec((_K, _D), lambda i: (0, 0)),
    ],
    out_specs=[
        pl.BlockSpec((1, 1, _R), lambda i: (i, 0, 0)),
        pl.BlockSpec((1, 1), lambda i: (0, 0)),
    ],
    out_shape=[
        jax.ShapeDtypeStruct((_G, 1, _R), jnp.int32),
        jax.ShapeDtypeStruct((1, 1), jnp.float32),
    ],
)


_sc_mesh = plsc.VectorSubcoreMesh(core_axis_name="c", subcore_axis_name="s")


@functools.partial(
    pl.kernel,
    out_type=jax.ShapeDtypeStruct((_N * _D,), jnp.float32),
    mesh=_sc_mesh,
    scratch_types=[
        pltpu.VMEM((_IC,), jnp.int32),
        pltpu.VMEM((_IC,), jnp.int32),
        pltpu.VMEM((_RW, 128), jnp.float32),
        pltpu.VMEM((_RW * _D,), jnp.float32),
        pltpu.SemaphoreType.DMA,
    ],
)
def _sc_gather(e_hbm, idx_hbm, x_hbm, out_hbm, idx_v0, idx_v1, q_v, x_v, sem):
    wid = lax.axis_index("s") * _NC + lax.axis_index("c")
    base = wid * _RW
    fbase = wid * (_RW * _D)
    pltpu.sync_copy(idx_hbm.at[pl.ds(base, _IC)], idx_v0)
    pltpu.sync_copy(idx_hbm.at[pl.ds(base + _IC, _IC)], idx_v1)
    pltpu.sync_copy(x_hbm.at[pl.ds(fbase, _RW * _D)], x_v)
    cp0 = pltpu.async_copy(e_hbm.at[idx_v0], q_v.at[pl.ds(0, _IC), :], sem)
    cp1 = pltpu.async_copy(e_hbm.at[idx_v1], q_v.at[pl.ds(_IC, _IC), :], sem)
    cp0.wait()
    cp1.wait()

    def body(i, _):
        xa = x_v[pl.ds(i * _D, 16)]
        qa = q_v[i, pl.ds(0, 16)]
        x_v[pl.ds(i * _D, 16)] = xa + (qa - xa)
        xb = x_v[pl.ds(i * _D + 16, 16)]
        qb = q_v[i, pl.ds(16, 16)]
        x_v[pl.ds(i * _D + 16, 16)] = xb + (qb - xb)
        return 0

    lax.fori_loop(0, _RW, body, 0)
    pltpu.sync_copy(x_v, out_hbm.at[pl.ds(fbase, _RW * _D)])


def kernel(inputs, embedding_weight):
    x = jnp.transpose(inputs, (0, 2, 3, 1)).reshape(_N, _D)
    a = jnp.sum(x ** 2, axis=1, keepdims=True)
    b = jnp.sum(embedding_weight ** 2, axis=1).reshape(1, _K)
    idx3, acc = _tc_call(x, a, b, embedding_weight)
    idx = idx3.reshape(_N)
    e_pad = jnp.pad(embedding_weight, ((0, 0), (0, 128 - _D)))
    out_flat = _sc_gather(e_pad, idx, x.reshape(_N * _D))
    loss = (1.0 + _BETA) * acc[0, 0] / jnp.float32(_N * _D)
    out = out_flat.reshape(8, 32, 32, _D).transpose(0, 3, 1, 2)
    return (loss, out)
